# baseline (speedup 1.0000x reference)
import math

import jax
import jax.numpy as jnp
from jax.experimental import pallas as pl
from jax.experimental.pallas import tpu as pltpu

B = 64
C = 3
H = 224
W = 224
HW = H * W
D_MODEL = 1024
N_EXPERTS = 6
NUM_CLASSES = 10
CAE_EXPERT_IDX = 5
ENTROPY_THRESHOLD = math.log(5) / 2.0

BB = 2
NCHUNK = B // BB
NBUF = 8


def _moe_kernel(x_ref, wb_ref, bb_ref, wg_ref, bg_ref, we_ref, be_ref,
                logits_ref, eid_ref, gates_ref, ent_ref, ood_ref,
                acc_ref, buf_ref, sem_ref):
    def start(k):
        pltpu.make_async_copy(
            x_ref.at[pl.ds(k * BB, BB)],
            buf_ref.at[k % NBUF],
            sem_ref.at[k % NBUF],
        ).start()

    for k in range(NBUF):
        start(k)
    for k in range(NCHUNK):
        pltpu.make_async_copy(
            x_ref.at[pl.ds(k * BB, BB)],
            buf_ref.at[k % NBUF],
            sem_ref.at[k % NBUF],
        ).wait()
        acc_ref[pl.ds(k * BB, BB), :] = jnp.sum(buf_ref[k % NBUF], axis=(2, 3))
        if k + NBUF < NCHUNK:
            start(k + NBUF)

    if True:
        pooled = acc_ref[...] * (1.0 / HW)
        wb = wb_ref[...]
        z = (pooled[:, 0:1] * wb[0:1, :]
             + pooled[:, 1:2] * wb[1:2, :]
             + pooled[:, 2:3] * wb[2:3, :]) + bb_ref[...]
        glog = jax.lax.dot_general(
            z, wg_ref[...], (((1,), (0,)), ((), ())),
            preferred_element_type=jnp.float32) + bg_ref[...]
        m = jnp.max(glog, axis=1, keepdims=True)
        e = jnp.exp(glog - m)
        g = e / jnp.sum(e, axis=1, keepdims=True)
        ent = -jnp.sum(g * jnp.log(g + 1e-8), axis=1, keepdims=True)
        ood = ent > ENTROPY_THRESHOLD
        gmax = jnp.max(g, axis=1, keepdims=True)
        gi = jax.lax.broadcasted_iota(jnp.int32, (B, 5), 1)
        dom = jnp.min(jnp.where(g >= gmax, gi, 5), axis=1, keepdims=True)
        eid = jnp.where(ood, CAE_EXPERT_IDX, dom).astype(jnp.int32)
        all6 = jax.lax.dot_general(
            z, we_ref[...], (((1,), (0,)), ((), ())),
            preferred_element_type=jnp.float32) + be_ref[...]
        out = jnp.zeros((B, NUM_CLASSES), jnp.float32)
        for ex in range(N_EXPERTS):
            sel = eid == ex
            out = out + jnp.where(sel, all6[:, 10 * ex:10 * ex + 10], 0.0)
        logits_ref[...] = out
        eid_ref[...] = eid
        gates_ref[...] = g
        ent_ref[...] = ent
        ood_ref[...] = ood.astype(jnp.int32)


def kernel(x, W_backbone, b_backbone, W_gate, b_gate, W_experts, b_experts):
    we2 = W_experts.transpose(1, 0, 2).reshape(D_MODEL, N_EXPERTS * NUM_CLASSES)
    be2 = b_experts.reshape(1, N_EXPERTS * NUM_CLASSES)
    outs = pl.pallas_call(
        _moe_kernel,
        in_specs=[
            pl.BlockSpec(memory_space=pl.ANY),
            pl.BlockSpec((C, D_MODEL), lambda: (0, 0)),
            pl.BlockSpec((1, D_MODEL), lambda: (0, 0)),
            pl.BlockSpec((D_MODEL, 5), lambda: (0, 0)),
            pl.BlockSpec((1, 5), lambda: (0, 0)),
            pl.BlockSpec((D_MODEL, N_EXPERTS * NUM_CLASSES), lambda: (0, 0)),
            pl.BlockSpec((1, N_EXPERTS * NUM_CLASSES), lambda: (0, 0)),
        ],
        out_specs=[
            pl.BlockSpec((B, NUM_CLASSES), lambda: (0, 0)),
            pl.BlockSpec((B, 1), lambda: (0, 0)),
            pl.BlockSpec((B, 5), lambda: (0, 0)),
            pl.BlockSpec((B, 1), lambda: (0, 0)),
            pl.BlockSpec((B, 1), lambda: (0, 0)),
        ],
        out_shape=[
            jax.ShapeDtypeStruct((B, NUM_CLASSES), jnp.float32),
            jax.ShapeDtypeStruct((B, 1), jnp.int32),
            jax.ShapeDtypeStruct((B, 5), jnp.float32),
            jax.ShapeDtypeStruct((B, 1), jnp.float32),
            jax.ShapeDtypeStruct((B, 1), jnp.int32),
        ],
        scratch_shapes=[
            pltpu.VMEM((B, C), jnp.float32),
            pltpu.VMEM((NBUF, BB, C, H, W), jnp.float32),
            pltpu.SemaphoreType.DMA((NBUF,)),
        ],
    )(x, W_backbone, b_backbone.reshape(1, D_MODEL), W_gate,
      b_gate.reshape(1, 5), we2, be2)
    logits, eid, gates, ent, ood = outs
    return (logits, eid[:, 0], gates, ent[:, 0], ood[:, 0].astype(jnp.bool_))


# speedup vs baseline: 1.1501x; 1.1501x over previous
import math

import jax
import jax.numpy as jnp
from jax.experimental import pallas as pl
from jax.experimental.pallas import tpu as pltpu

B = 64
C = 3
H = 224
W = 224
HW = H * W
D_MODEL = 1024
N_EXPERTS = 6
NUM_CLASSES = 10
CAE_EXPERT_IDX = 5
ENTROPY_THRESHOLD = math.log(5) / 2.0

BB = 8
GRID = B // BB


def _moe_kernel(x_ref, wb_ref, bb_ref, wg_ref, bg_ref, we_ref, be_ref,
                logits_ref, eid_ref, gates_ref, ent_ref, ood_ref, acc_ref):
    i = pl.program_id(0)
    part = jnp.sum(x_ref[...], axis=(2, 3))
    acc_ref[pl.ds(i * BB, BB), :] = part

    @pl.when(i == GRID - 1)
    def _epilogue():
        pooled = acc_ref[...] * (1.0 / HW)
        wb = wb_ref[...]
        z = (pooled[:, 0:1] * wb[0:1, :]
             + pooled[:, 1:2] * wb[1:2, :]
             + pooled[:, 2:3] * wb[2:3, :]) + bb_ref[...][None, :]
        glog = jax.lax.dot_general(
            z, wg_ref[...], (((1,), (0,)), ((), ())),
            preferred_element_type=jnp.float32) + bg_ref[...][None, :]
        m = jnp.max(glog, axis=1, keepdims=True)
        e = jnp.exp(glog - m)
        g = e / jnp.sum(e, axis=1, keepdims=True)
        ent = -jnp.sum(g * jnp.log(g + 1e-8), axis=1, keepdims=True)
        ood = ent > ENTROPY_THRESHOLD
        gmax = jnp.max(g, axis=1, keepdims=True)
        gi = jax.lax.broadcasted_iota(jnp.int32, (B, 5), 1)
        dom = jnp.min(jnp.where(g >= gmax, gi, 5), axis=1, keepdims=True)
        eid = jnp.where(ood, CAE_EXPERT_IDX, dom).astype(jnp.int32)
        out = jnp.zeros((B, NUM_CLASSES), jnp.float32)
        for ex in range(N_EXPERTS):
            contrib = jax.lax.dot_general(
                z, we_ref[ex], (((1,), (0,)), ((), ())),
                preferred_element_type=jnp.float32) + be_ref[ex:ex + 1, :]
            out = out + jnp.where(eid == ex, contrib, 0.0)
        logits_ref[...] = out
        eid_ref[...] = eid.reshape(B)
        gates_ref[...] = g
        ent_ref[...] = ent.reshape(B)
        ood_ref[...] = ood.reshape(B)


def kernel(x, W_backbone, b_backbone, W_gate, b_gate, W_experts, b_experts):
    outs = pl.pallas_call(
        _moe_kernel,
        grid=(GRID,),
        in_specs=[
            pl.BlockSpec((BB, C, H, W), lambda i: (i, 0, 0, 0)),
            pl.BlockSpec((C, D_MODEL), lambda i: (0, 0)),
            pl.BlockSpec((D_MODEL,), lambda i: (0,)),
            pl.BlockSpec((D_MODEL, 5), lambda i: (0, 0)),
            pl.BlockSpec((5,), lambda i: (0,)),
            pl.BlockSpec((N_EXPERTS, D_MODEL, NUM_CLASSES), lambda i: (0, 0, 0)),
            pl.BlockSpec((N_EXPERTS, NUM_CLASSES), lambda i: (0, 0)),
        ],
        out_specs=[
            pl.BlockSpec((B, NUM_CLASSES), lambda i: (0, 0)),
            pl.BlockSpec((B,), lambda i: (0,)),
            pl.BlockSpec((B, 5), lambda i: (0, 0)),
            pl.BlockSpec((B,), lambda i: (0,)),
            pl.BlockSpec((B,), lambda i: (0,)),
        ],
        out_shape=[
            jax.ShapeDtypeStruct((B, NUM_CLASSES), jnp.float32),
            jax.ShapeDtypeStruct((B,), jnp.int32),
            jax.ShapeDtypeStruct((B, 5), jnp.float32),
            jax.ShapeDtypeStruct((B,), jnp.float32),
            jax.ShapeDtypeStruct((B,), jnp.bool_),
        ],
        scratch_shapes=[pltpu.VMEM((B, C), jnp.float32)],
    )(x, W_backbone, b_backbone, W_gate, b_gate, W_experts, b_experts)
    return tuple(outs)


# speedup vs baseline: 1.5589x; 1.3555x over previous
import math

import jax
import jax.numpy as jnp
from jax.experimental import pallas as pl
from jax.experimental.pallas import tpu as pltpu

B = 64
C = 3
H = 224
W = 224
HW = H * W
D_MODEL = 1024
N_EXPERTS = 6
NUM_CLASSES = 10
CAE_EXPERT_IDX = 5
ENTROPY_THRESHOLD = math.log(5) / 2.0

BB = 8
GRID = B // BB


def _moe_kernel(x_ref, wb_ref, bb_ref, pk_ref, bg_ref, we_ref, be_ref,
                logits_ref, eid_ref, gates_ref, ent_ref, ood_ref, acc_ref):
    i = pl.program_id(0)
    part = jnp.sum(x_ref[...], axis=(2, 3))
    acc_ref[pl.ds(i * BB, BB), :] = part

    @pl.when(i == GRID - 1)
    def _epilogue():
        pooled = acc_ref[...] * (1.0 / HW)
        wb = wb_ref[...]
        z = (pooled[:, 0:1] * wb[0:1, :]
             + pooled[:, 1:2] * wb[1:2, :]
             + pooled[:, 2:3] * wb[2:3, :]) + bb_ref[...][None, :]
        glog = jax.lax.dot_general(
            z, pk_ref[0:5, :], (((1,), (1,)), ((), ())),
            preferred_element_type=jnp.float32) + bg_ref[...][None, :]
        m = jnp.max(glog, axis=1, keepdims=True)
        e = jnp.exp(glog - m)
        g = e / jnp.sum(e, axis=1, keepdims=True)
        ent = -jnp.sum(g * jnp.log(g + 1e-8), axis=1, keepdims=True)
        ood = ent > ENTROPY_THRESHOLD
        gmax = jnp.max(g, axis=1, keepdims=True)
        gi = jax.lax.broadcasted_iota(jnp.int32, (B, 5), 1)
        dom = jnp.min(jnp.where(g >= gmax, gi, 5), axis=1, keepdims=True)
        eid = jnp.where(ood, CAE_EXPERT_IDX, dom).astype(jnp.int32)
        out = jnp.zeros((B, NUM_CLASSES), jnp.float32)
        for ex in range(N_EXPERTS):
            contrib = jax.lax.dot_general(
                z, we_ref[:, D_MODEL * ex:D_MODEL * (ex + 1)],
                (((1,), (1,)), ((), ())),
                preferred_element_type=jnp.float32) + be_ref[ex:ex + 1, :]
            out = out + jnp.where(eid == ex, contrib, 0.0)
        logits_ref[...] = out.T
        eid_ref[...] = eid.reshape(B)
        gates_ref[...] = g.T
        ent_ref[...] = ent.reshape(B)
        ood_ref[...] = ood.reshape(B)


def kernel(x, W_backbone, b_backbone, W_gate, b_gate, W_experts, b_experts):
    wgt = W_gate.T
    wet = W_experts.reshape(N_EXPERTS * D_MODEL, NUM_CLASSES).T
    outs = pl.pallas_call(
        _moe_kernel,
        grid=(GRID,),
        in_specs=[
            pl.BlockSpec((BB, C, H, W), lambda i: (i, 0, 0, 0)),
            pl.BlockSpec((C, D_MODEL), lambda i: (0, 0)),
            pl.BlockSpec((D_MODEL,), lambda i: (0,)),
            pl.BlockSpec((5, D_MODEL), lambda i: (0, 0)),
            pl.BlockSpec((5,), lambda i: (0,)),
            pl.BlockSpec((NUM_CLASSES, N_EXPERTS * D_MODEL), lambda i: (0, 0)),
            pl.BlockSpec((N_EXPERTS, NUM_CLASSES), lambda i: (0, 0)),
        ],
        out_specs=[
            pl.BlockSpec((NUM_CLASSES, B), lambda i: (0, 0)),
            pl.BlockSpec((B,), lambda i: (0,)),
            pl.BlockSpec((5, B), lambda i: (0, 0)),
            pl.BlockSpec((B,), lambda i: (0,)),
            pl.BlockSpec((B,), lambda i: (0,)),
        ],
        out_shape=[
            jax.ShapeDtypeStruct((NUM_CLASSES, B), jnp.float32),
            jax.ShapeDtypeStruct((B,), jnp.int32),
            jax.ShapeDtypeStruct((5, B), jnp.float32),
            jax.ShapeDtypeStruct((B,), jnp.float32),
            jax.ShapeDtypeStruct((B,), jnp.bool_),
        ],
        scratch_shapes=[pltpu.VMEM((B, C), jnp.float32)],
    )(x, W_backbone, b_backbone, wgt, b_gate, wet, b_experts)
    logits_t, eid, gates_t, ent, ood = outs
    return (logits_t.T, eid, gates_t.T, ent, ood)
